# baseline (speedup 1.0000x reference)
import functools

import jax
import jax.numpy as jnp
from jax.experimental import pallas as pl
from jax.experimental.pallas import tpu as pltpu
from jax.experimental.pallas import tpu_sc as plsc

_B = 1024
_C = 100000

_SC_LO = 59392
_SC_HI = 98304
_SC_BLOCKS = (_SC_HI - _SC_LO) // 2048
_SC_CHUNKS = (_SC_HI - _SC_LO) // 512
_SC_OUT_W = _SC_BLOCKS * 128

_TC_ROWS = 32
_TC_CHUNKS = 4
_TC_W = _SC_LO // _TC_CHUNKS
_TAIL_BLOCK = _SC_HI // 2048
_TAIL_VALID = _C - _SC_HI

_WIN = 512
_CMB_ROWS = 128

_NEG = float("-inf")


def _sc_block_max(prediction):
    mesh = plsc.VectorSubcoreMesh(core_axis_name="c", subcore_axis_name="s")

    @pl.kernel(
        out_type=jax.ShapeDtypeStruct((_B, _SC_OUT_W), jnp.float32),
        mesh=mesh,
    )
    def sc_kernel(p_hbm, o_hbm):
        def body(in_vmem, out_vmem):
            neg = jnp.full((16,), _NEG, jnp.float32)

            @pl.loop(0, 8)
            def _(r):
                @pl.loop(0, 4)
                def _(j):
                    base = j * _WIN
                    acc = functools.reduce(
                        jnp.maximum,
                        [
                            in_vmem[r, pl.ds(base + 16 * s, 16)]
                            for s in range(_WIN // 16)
                        ],
                    )
                    out_vmem[r, pl.ds(j * 16, 16)] = acc
                    out_vmem[r, pl.ds(64 + j * 16, 16)] = neg

        pltpu.emit_pipeline(
            body,
            grid=(_B // 8, _SC_BLOCKS),
            in_specs=[
                pl.BlockSpec((8, 2048), index_map=lambda i, j: (i, _SC_LO // 2048 + j))
            ],
            out_specs=[pl.BlockSpec((8, 128), index_map=lambda i, j: (i, j))],
            core_axis_name=("c", "s"),
            dimension_semantics=(pltpu.PARALLEL, pltpu.PARALLEL),
        )(p_hbm, o_hbm)

    return sc_kernel(prediction)


def _tc_block(y_smem, *refs):
    chunk_refs = refs[:_TC_CHUNKS]
    tail_ref = refs[_TC_CHUNKS]
    win_refs = refs[_TC_CHUNKS + 1 : _TC_CHUNKS + 1 + _TC_ROWS]
    yv_ref = refs[_TC_CHUNKS + 1 + _TC_ROWS]
    tcm_ref, cls_ref, wm_ref = refs[_TC_CHUNKS + 2 + _TC_ROWS :]

    yv = yv_ref[...]
    col = jax.lax.broadcasted_iota(jnp.int32, (_TC_ROWS, _TC_W), 1)
    tcm = None
    for k, p_ref in enumerate(chunk_refs):
        bad = col == (yv - k * _TC_W)
        mk = jnp.max(jnp.where(bad, _NEG, p_ref[...]), axis=1, keepdims=True)
        tcm = mk if tcm is None else jnp.maximum(tcm, mk)

    col_t = jax.lax.broadcasted_iota(jnp.int32, (_TC_ROWS, 2048), 1)
    bad_t = (col_t == (yv - _SC_HI)) | (col_t >= _TAIL_VALID)
    mt = jnp.max(jnp.where(bad_t, _NEG, tail_ref[...]), axis=1, keepdims=True)
    tcm_ref[...] = jnp.maximum(tcm, mt)

    win = jnp.concatenate(
        [w_ref[r % 8 : r % 8 + 1, :] for r, w_ref in enumerate(win_refs)], axis=0
    )
    lane = jax.lax.broadcasted_iota(jnp.int32, (_TC_ROWS, _WIN), 1)
    qv = yv // _WIN
    is_y = lane == (yv - qv * _WIN)
    oob = (qv * _WIN + lane) >= _C
    cls_ref[...] = jnp.max(jnp.where(is_y, win, _NEG), axis=1, keepdims=True)
    wm_ref[...] = jnp.max(jnp.where(is_y | oob, _NEG, win), axis=1, keepdims=True)


def _combine_block(sc_ref, yv_ref, tcm_ref, cls_ref, wm_ref, out_ref):
    yv = yv_ref[...]
    pos = jax.lax.broadcasted_iota(jnp.int32, (_CMB_ROWS, _SC_OUT_W), 1)
    j_y = (yv - _SC_LO) // _WIN
    in_sc = (yv >= _SC_LO) & (yv < _SC_HI)
    chunk_of_pos = (pos // 128) * 4 + (pos % 128) // 16
    drop = in_sc & (chunk_of_pos == j_y)
    sc_masked = jnp.max(
        jnp.where(drop, _NEG, sc_ref[...]), axis=1, keepdims=True
    )
    target = jnp.maximum(jnp.maximum(sc_masked, tcm_ref[...]), wm_ref[...])
    out_ref[...] = target - cls_ref[...]


def kernel(prediction, y):
    batch, num_classes = prediction.shape
    assert (batch, num_classes) == (_B, _C)
    y32 = y.astype(jnp.int32)
    y2 = y32.reshape(batch, 1)

    sc_out = _sc_block_max(prediction)

    r = _TC_ROWS
    chunk_specs = [
        pl.BlockSpec((r, _TC_W), lambda i, ys, kk=kk: (i, kk))
        for kk in range(_TC_CHUNKS)
    ]
    tail_spec = pl.BlockSpec((r, 2048), lambda i, ys: (i, _TAIL_BLOCK))
    win_specs = [
        pl.BlockSpec(
            (8, _WIN),
            lambda i, ys, rr=rr: ((i * _TC_ROWS + rr) // 8, ys[i * _TC_ROWS + rr] // _WIN),
        )
        for rr in range(r)
    ]
    grid_spec = pltpu.PrefetchScalarGridSpec(
        num_scalar_prefetch=1,
        grid=(batch // r,),
        in_specs=chunk_specs
        + [tail_spec]
        + win_specs
        + [pl.BlockSpec((r, 1), lambda i, ys: (i, 0))],
        out_specs=[
            pl.BlockSpec((r, 1), lambda i, ys: (i, 0)),
            pl.BlockSpec((r, 1), lambda i, ys: (i, 0)),
            pl.BlockSpec((r, 1), lambda i, ys: (i, 0)),
        ],
    )
    tcm, cls, wm = pl.pallas_call(
        _tc_block,
        grid_spec=grid_spec,
        out_shape=[
            jax.ShapeDtypeStruct((batch, 1), jnp.float32),
            jax.ShapeDtypeStruct((batch, 1), jnp.float32),
            jax.ShapeDtypeStruct((batch, 1), jnp.float32),
        ],
    )(y32, *([prediction] * (_TC_CHUNKS + 1 + r)), y2)

    rc = _CMB_ROWS
    out = pl.pallas_call(
        _combine_block,
        grid=(batch // rc,),
        in_specs=[
            pl.BlockSpec((rc, _SC_OUT_W), lambda i: (i, 0)),
            pl.BlockSpec((rc, 1), lambda i: (i, 0)),
            pl.BlockSpec((rc, 1), lambda i: (i, 0)),
            pl.BlockSpec((rc, 1), lambda i: (i, 0)),
            pl.BlockSpec((rc, 1), lambda i: (i, 0)),
        ],
        out_specs=pl.BlockSpec((rc, 1), lambda i: (i, 0)),
        out_shape=jax.ShapeDtypeStruct((batch, 1), jnp.float32),
    )(sc_out, y2, tcm, cls, wm)
    return out.reshape(batch)
